# Initial kernel scaffold; baseline (speedup 1.0000x reference)
import jax
import jax.numpy as jnp
from jax.experimental import pallas as pl


def kernel(logits, targets):
    raise NotImplementedError("write your pallas kernel here")


# speedup vs baseline: 38.3100x; 38.3100x over previous
import functools

import jax
import jax.numpy as jnp
from jax import lax
from jax.experimental import pallas as pl
from jax.experimental.pallas import tpu as pltpu
from jax.experimental.pallas import tpu_sc as plsc

_ALPHA = 0.7
_C = 19
_K = 1024
_NBINS = 2 * _C * _K
_NC, _NS = 2, 16
_NW = _NC * _NS
_CHUNK = 32768


def _prep_body(logits_ref, targets_ref, idx_ref, ce_ref):
    b = pl.program_id(0)
    r = pl.program_id(1)

    l = logits_ref[0]
    t = targets_ref[0]

    m = jnp.max(l, axis=0)
    ex = jnp.exp(l - m[None, :, :])
    s = jnp.sum(ex, axis=0)
    lse = m + jnp.log(s)

    cls = lax.broadcasted_iota(jnp.int32, (_C, 8, 512), 0)
    fg = (t[None, :, :] == cls)

    p = ex / s[None, :, :]
    e = jnp.where(fg, 1.0 - p, p)
    bucket = jnp.minimum((e * float(_K)).astype(jnp.int32), _K - 1)
    idx_ref[0] = (jnp.where(fg, _C * _K, 0) + cls * _K) + bucket

    l_t = jnp.sum(jnp.where(fg, l, 0.0), axis=0)
    part = jnp.sum((lse - l_t).reshape(8, 4, 128), axis=1)

    @pl.when((b == 0) & (r == 0))
    def _():
        ce_ref[...] = jnp.zeros_like(ce_ref)

    ce_ref[...] += part


def _sc_hist_body(idx_hbm, out_hbm, chunk_v, hist_v):
    wid = lax.axis_index("s") * _NC + lax.axis_index("c")
    per_tile = idx_hbm.shape[0] // _NW
    n_chunks = per_tile // _CHUNK
    base = wid * per_tile

    def zero_body(i, _):
        hist_v[pl.ds(i * 16, 16)] = jnp.zeros((16,), jnp.int32)
        return 0

    lax.fori_loop(0, _NBINS // 16, zero_body, 0, unroll=8)

    def chunk_body(j, _):
        pltpu.sync_copy(idx_hbm.at[pl.ds(base + j * _CHUNK, _CHUNK)], chunk_v)

        def vec_body(i, _):
            v = chunk_v[pl.ds(i * 16, 16)]
            cnt, last = plsc.scan_count(v)
            plsc.addupdate_scatter(hist_v, [v], cnt, mask=last)
            return 0

        lax.fori_loop(0, _CHUNK // 16, vec_body, 0, unroll=8)
        return 0

    lax.fori_loop(0, n_chunks, chunk_body, 0)
    pltpu.sync_copy(hist_v, out_hbm.at[wid])


def _final_body(hist_ref, ce_ref, out_ref, *, n_pix):
    h = hist_ref[...].astype(jnp.float32)
    acc = jnp.sum(h, axis=0)
    n = acc[0:_C, :] + acc[_C:2 * _C, :]
    f = acc[_C:2 * _C, :]

    row = lax.broadcasted_iota(jnp.int32, (_K, _K), 0)
    col = lax.broadcasted_iota(jnp.int32, (_K, _K), 1)
    m_tri = (row >= col).astype(jnp.float32)

    p_su = jnp.dot(n, m_tri, preferred_element_type=jnp.float32)
    f_su = jnp.dot(f, m_tri, preferred_element_type=jnp.float32)
    g = f_su[:, 0:1]

    jac = jnp.where(p_su > 0.0, 1.0 - (g - f_su) / (g + p_su - f_su), 0.0)
    dots = (jnp.sum(jac, axis=1) - 0.5) / float(_K)
    loss_lv = jnp.sum(dots) / float(_C)

    ce = jnp.sum(ce_ref[...]) / float(n_pix)

    out_ref[0, 0] = _ALPHA * loss_lv + (1.0 - _ALPHA) * ce


def kernel(logits, targets):
    B, C, H, W = logits.shape
    n_total = B * C * H * W

    idx, ce = pl.pallas_call(
        _prep_body,
        grid=(B, H // 8),
        in_specs=[
            pl.BlockSpec((1, C, 8, W), lambda b, r: (b, 0, r, 0)),
            pl.BlockSpec((1, 8, W), lambda b, r: (b, r, 0)),
        ],
        out_specs=[
            pl.BlockSpec((1, C, 8, W), lambda b, r: (b, 0, r, 0)),
            pl.BlockSpec((8, 128), lambda b, r: (0, 0)),
        ],
        out_shape=[
            jax.ShapeDtypeStruct((B, C, H, W), jnp.int32),
            jax.ShapeDtypeStruct((8, 128), jnp.float32),
        ],
    )(logits, targets)

    mesh = plsc.VectorSubcoreMesh(core_axis_name="c", subcore_axis_name="s")
    hist = pl.kernel(
        _sc_hist_body,
        out_type=jax.ShapeDtypeStruct((_NW, _NBINS), jnp.int32),
        mesh=mesh,
        scratch_types=[
            pltpu.VMEM((_CHUNK,), jnp.int32),
            pltpu.VMEM((_NBINS,), jnp.int32),
        ],
        compiler_params=pltpu.CompilerParams(needs_layout_passes=False),
    )(idx.reshape(n_total))

    out = pl.pallas_call(
        functools.partial(_final_body, n_pix=B * H * W),
        out_specs=pl.BlockSpec(memory_space=pltpu.SMEM),
        out_shape=jax.ShapeDtypeStruct((1, 1), jnp.float32),
    )(hist.reshape(_NW, 2 * _C, _K), ce)

    return out[0, 0]


# speedup vs baseline: 64.2197x; 1.6763x over previous
import functools

import jax
import jax.numpy as jnp
from jax import lax
from jax.experimental import pallas as pl
from jax.experimental.pallas import tpu as pltpu
from jax.experimental.pallas import tpu_sc as plsc

_ALPHA = 0.7
_C = 19
_K = 1024
_NBINS = 2 * _C * _K
_NC, _NS = 2, 16
_NW = _NC * _NS
_CHUNK = 32768


def _prep_body(logits_ref, targets_ref, idx_ref, ce_ref):
    b = pl.program_id(0)
    r = pl.program_id(1)

    l = logits_ref[0]
    t = targets_ref[0]

    m = jnp.max(l, axis=0)
    ex = jnp.exp(l - m[None, :, :])
    s = jnp.sum(ex, axis=0)
    lse = m + jnp.log(s)

    cls = lax.broadcasted_iota(jnp.int32, (_C, 8, 512), 0)
    fg = (t[None, :, :] == cls)

    p = ex / s[None, :, :]
    e = jnp.where(fg, 1.0 - p, p)
    bucket = jnp.minimum((e * float(_K)).astype(jnp.int32), _K - 1)
    idx_ref[0] = (jnp.where(fg, _C * _K, 0) + cls * _K) + bucket

    l_t = jnp.sum(jnp.where(fg, l, 0.0), axis=0)
    part = jnp.sum((lse - l_t).reshape(8, 4, 128), axis=1)

    @pl.when((b == 0) & (r == 0))
    def _():
        ce_ref[...] = jnp.zeros_like(ce_ref)

    ce_ref[...] += part


def _sc_hist_body(idx_hbm, out_hbm, chunk_v, hist_v):
    wid = lax.axis_index("s") * _NC + lax.axis_index("c")
    per_tile = idx_hbm.shape[0] // _NW
    n_chunks = per_tile // _CHUNK
    base = wid * per_tile

    def zero_body(i, _):
        hist_v[pl.ds(i * 16, 16)] = jnp.zeros((16,), jnp.int32)
        return 0

    lax.fori_loop(0, _NBINS // 16, zero_body, 0, unroll=8)

    def chunk_body(j, _):
        pltpu.sync_copy(idx_hbm.at[pl.ds(base + j * _CHUNK, _CHUNK)], chunk_v)

        ones = jnp.ones((16,), jnp.int32)

        def vec_body(i, _):
            v = chunk_v[pl.ds(i * 16, 16)]
            plsc.addupdate_scatter(hist_v, [v], ones)
            return 0

        lax.fori_loop(0, _CHUNK // 16, vec_body, 0, unroll=8)
        return 0

    lax.fori_loop(0, n_chunks, chunk_body, 0)
    pltpu.sync_copy(hist_v, out_hbm.at[wid])


def _final_body(hist_ref, ce_ref, out_ref, *, n_pix):
    h = hist_ref[...].astype(jnp.float32)
    acc = jnp.sum(h, axis=0)
    n = acc[0:_C, :] + acc[_C:2 * _C, :]
    f = acc[_C:2 * _C, :]

    row = lax.broadcasted_iota(jnp.int32, (_K, _K), 0)
    col = lax.broadcasted_iota(jnp.int32, (_K, _K), 1)
    m_tri = (row >= col).astype(jnp.float32)

    p_su = jnp.dot(n, m_tri, preferred_element_type=jnp.float32)
    f_su = jnp.dot(f, m_tri, preferred_element_type=jnp.float32)
    g = f_su[:, 0:1]

    jac = jnp.where(p_su > 0.0, 1.0 - (g - f_su) / (g + p_su - f_su), 0.0)
    dots = (jnp.sum(jac, axis=1) - 0.5) / float(_K)
    loss_lv = jnp.sum(dots) / float(_C)

    ce = jnp.sum(ce_ref[...]) / float(n_pix)

    out_ref[0, 0] = _ALPHA * loss_lv + (1.0 - _ALPHA) * ce


def kernel(logits, targets):
    B, C, H, W = logits.shape
    n_total = B * C * H * W

    idx, ce = pl.pallas_call(
        _prep_body,
        grid=(B, H // 8),
        in_specs=[
            pl.BlockSpec((1, C, 8, W), lambda b, r: (b, 0, r, 0)),
            pl.BlockSpec((1, 8, W), lambda b, r: (b, r, 0)),
        ],
        out_specs=[
            pl.BlockSpec((1, C, 8, W), lambda b, r: (b, 0, r, 0)),
            pl.BlockSpec((8, 128), lambda b, r: (0, 0)),
        ],
        out_shape=[
            jax.ShapeDtypeStruct((B, C, H, W), jnp.int32),
            jax.ShapeDtypeStruct((8, 128), jnp.float32),
        ],
    )(logits, targets)

    mesh = plsc.VectorSubcoreMesh(core_axis_name="c", subcore_axis_name="s")
    hist = pl.kernel(
        _sc_hist_body,
        out_type=jax.ShapeDtypeStruct((_NW, _NBINS), jnp.int32),
        mesh=mesh,
        scratch_types=[
            pltpu.VMEM((_CHUNK,), jnp.int32),
            pltpu.VMEM((_NBINS,), jnp.int32),
        ],
        compiler_params=pltpu.CompilerParams(needs_layout_passes=False),
    )(idx.reshape(n_total))

    out = pl.pallas_call(
        functools.partial(_final_body, n_pix=B * H * W),
        out_specs=pl.BlockSpec(memory_space=pltpu.SMEM),
        out_shape=jax.ShapeDtypeStruct((1, 1), jnp.float32),
    )(hist.reshape(_NW, 2 * _C, _K), ce)

    return out[0, 0]


# speedup vs baseline: 77.5383x; 1.2074x over previous
import functools

import jax
import jax.numpy as jnp
from jax import lax
from jax.experimental import pallas as pl
from jax.experimental.pallas import tpu as pltpu
from jax.experimental.pallas import tpu_sc as plsc

_ALPHA = 0.7
_C = 19
_K = 1024
_NBINS = 2 * _C * _K
_NC, _NS = 2, 16
_NW = _NC * _NS
_CHUNK = 32768


def _prep_body(logits_ref, targets_ref, idx_ref, ce_ref):
    b = pl.program_id(0)
    r = pl.program_id(1)

    l = logits_ref[0]
    t = targets_ref[0]

    m = jnp.max(l, axis=0)
    ex = jnp.exp(l - m[None, :, :])
    s = jnp.sum(ex, axis=0)
    lse = m + jnp.log(s)

    cls = lax.broadcasted_iota(jnp.int32, (_C, 8, 512), 0)
    fg = (t[None, :, :] == cls)

    p = ex / s[None, :, :]
    e = jnp.where(fg, 1.0 - p, p)
    bucket = jnp.minimum((e * float(_K)).astype(jnp.int32), _K - 1)
    idx_ref[0] = (jnp.where(fg, _C * _K, 0) + cls * _K) + bucket

    l_t = jnp.sum(jnp.where(fg, l, 0.0), axis=0)
    part = jnp.sum((lse - l_t).reshape(8, 4, 128), axis=1)

    @pl.when((b == 0) & (r == 0))
    def _():
        ce_ref[...] = jnp.zeros_like(ce_ref)

    ce_ref[...] += part


_ROWS = 64


def _sc_hist_body(idx_hbm, out_hbm, chunk0_v, chunk1_v, hist_v, sem0, sem1):
    wid = lax.axis_index("s") * _NC + lax.axis_index("c")
    rows_per_tile = idx_hbm.shape[0] // _NW
    n_chunks = rows_per_tile // _ROWS
    row0 = wid * rows_per_tile

    def zero_body(i, _):
        hist_v[pl.ds(i * 16, 16)] = jnp.zeros((16,), jnp.int32)
        return 0

    lax.fori_loop(0, _NBINS // 16, zero_body, 0, unroll=8)

    bufs = (chunk0_v, chunk1_v)
    sems = (sem0, sem1)

    def copy_in(j, slot):
        return pltpu.make_async_copy(
            idx_hbm.at[pl.ds(row0 + j * _ROWS, _ROWS), :], bufs[slot], sems[slot]
        )

    copy_in(0, 0).start()
    ones = jnp.ones((16,), jnp.int32)

    for j in range(n_chunks):
        slot = j % 2
        if j + 1 < n_chunks:
            copy_in(j + 1, 1 - slot).start()
        copy_in(j, slot).wait()
        buf = bufs[slot]

        def row_body(r, _):
            for k in range(32):
                v = buf[r, pl.ds(k * 16, 16)]
                plsc.addupdate_scatter(hist_v, [v], ones)
            return 0

        lax.fori_loop(0, _ROWS, row_body, 0)

    pltpu.sync_copy(hist_v, out_hbm.at[wid])


def _final_body(hist_ref, ce_ref, out_ref, *, n_pix):
    h = hist_ref[...].astype(jnp.float32)
    acc = jnp.sum(h, axis=0)
    n = acc[0:_C, :] + acc[_C:2 * _C, :]
    f = acc[_C:2 * _C, :]

    row = lax.broadcasted_iota(jnp.int32, (_K, _K), 0)
    col = lax.broadcasted_iota(jnp.int32, (_K, _K), 1)
    m_tri = (row >= col).astype(jnp.float32)

    p_su = jnp.dot(n, m_tri, preferred_element_type=jnp.float32)
    f_su = jnp.dot(f, m_tri, preferred_element_type=jnp.float32)
    g = f_su[:, 0:1]

    jac = jnp.where(p_su > 0.0, 1.0 - (g - f_su) / (g + p_su - f_su), 0.0)
    dots = (jnp.sum(jac, axis=1) - 0.5) / float(_K)
    loss_lv = jnp.sum(dots) / float(_C)

    ce = jnp.sum(ce_ref[...]) / float(n_pix)

    out_ref[0, 0] = _ALPHA * loss_lv + (1.0 - _ALPHA) * ce


def kernel(logits, targets):
    B, C, H, W = logits.shape
    n_total = B * C * H * W

    idx, ce = pl.pallas_call(
        _prep_body,
        grid=(B, H // 8),
        in_specs=[
            pl.BlockSpec((1, C, 8, W), lambda b, r: (b, 0, r, 0)),
            pl.BlockSpec((1, 8, W), lambda b, r: (b, r, 0)),
        ],
        out_specs=[
            pl.BlockSpec((1, C, 8, W), lambda b, r: (b, 0, r, 0)),
            pl.BlockSpec((8, 128), lambda b, r: (0, 0)),
        ],
        out_shape=[
            jax.ShapeDtypeStruct((B, C, H, W), jnp.int32),
            jax.ShapeDtypeStruct((8, 128), jnp.float32),
        ],
    )(logits, targets)

    mesh = plsc.VectorSubcoreMesh(core_axis_name="c", subcore_axis_name="s")
    hist = pl.kernel(
        _sc_hist_body,
        out_type=jax.ShapeDtypeStruct((_NW, _NBINS), jnp.int32),
        mesh=mesh,
        scratch_types=[
            pltpu.VMEM((_ROWS, W), jnp.int32),
            pltpu.VMEM((_ROWS, W), jnp.int32),
            pltpu.VMEM((_NBINS,), jnp.int32),
            pltpu.SemaphoreType.DMA,
            pltpu.SemaphoreType.DMA,
        ],
        compiler_params=pltpu.CompilerParams(needs_layout_passes=False),
    )(idx.reshape(n_total // W, W))

    out = pl.pallas_call(
        functools.partial(_final_body, n_pix=B * H * W),
        out_specs=pl.BlockSpec(memory_space=pltpu.SMEM),
        out_shape=jax.ShapeDtypeStruct((1, 1), jnp.float32),
    )(hist.reshape(_NW, 2 * _C, _K), ce)

    return out[0, 0]
